# baseline (speedup 1.0000x reference)
import jax
import jax.numpy as jnp
from jax import lax
from jax.experimental import pallas as pl
from jax.experimental.pallas import tpu as pltpu
from jax.experimental.pallas import tpu_sc as plsc

NC = 2
NS = 16
NW = NC * NS
CHUNK = 128
DEPTH = 4
ACHUNK = 64
DEG_W = 16


_SC_PARAMS = pltpu.CompilerParams(use_tc_tiling_on_sc=False)


def _mesh():
    return plsc.VectorSubcoreMesh(
        core_axis_name="c", subcore_axis_name="s",
        num_cores=NC, num_subcores=NS)


def _row_split(n, n_tiles):
    base = (-(-n // n_tiles) + 15) // 16 * 16
    offs, sizes = [], []
    for t in range(n_tiles):
        o = min(t * base, n)
        sz = min(base, n - o)
        offs.append(o)
        sizes.append(sz)
    return offs, sizes


def _deg_call(n, n_pad, e_pad):
    epw = e_pad // NW
    n_chunks = epw // CHUNK
    zrpt = n_pad // NS
    offs, sizes = _row_split(n, NS)

    def body(dst_hbm, zeros_hbm, ones_hbm, out,
             dst_v0, dst_v1, ones_v, deg_sh, isem0, isem1):
        c = lax.axis_index("c")
        s = lax.axis_index("s")
        wid = s * NC + c
        pltpu.sync_copy(
            zeros_hbm.at[pl.ds(s * zrpt, zrpt)],
            deg_sh.at[pl.ds(s * zrpt, zrpt)])
        pltpu.sync_copy(ones_hbm, ones_v)
        plsc.subcore_barrier()
        base = wid * epw

        pltpu.sync_copy(dst_hbm.at[pl.ds(base, CHUNK)], dst_v0)
        pltpu.async_copy(dst_hbm.at[pl.ds(base + CHUNK, CHUNK)], dst_v1, isem1)

        def half(i, dst_c, dst_n, isem_c, isem_n):
            @pl.when(i + 1 < n_chunks)
            def _():
                pltpu.make_async_copy(
                    dst_hbm.at[pl.ds(0, CHUNK)], dst_n, isem_n).wait()

            @pl.when(i + 2 < n_chunks)
            def _():
                pltpu.async_copy(
                    dst_hbm.at[pl.ds(base + (i + 2) * CHUNK, CHUNK)],
                    dst_c, isem_c)

            pltpu.sync_copy(ones_v, deg_sh.at[dst_c], add=True)

        def step(j, carry):
            half(2 * j, dst_v0, dst_v1, isem0, isem1)
            half(2 * j + 1, dst_v1, dst_v0, isem1, isem0)
            return carry

        lax.fori_loop(0, n_chunks // 2, step, 0)
        plsc.subcore_barrier()
        for t in range(NS):
            if sizes[t] > 0:
                @pl.when(s == t)
                def _():
                    pltpu.sync_copy(
                        deg_sh.at[pl.ds(offs[t], sizes[t])],
                        out.at[c, pl.ds(offs[t], sizes[t])])

    return pl.kernel(
        body,
        out_type=jax.ShapeDtypeStruct((NC, n, DEG_W), jnp.float32),
        mesh=_mesh(),
        compiler_params=_SC_PARAMS,
        scratch_types=[
            pltpu.VMEM((CHUNK,), jnp.int32),
            pltpu.VMEM((CHUNK,), jnp.int32),
            pltpu.VMEM((CHUNK, DEG_W), jnp.float32),
            pltpu.VMEM_SHARED((n_pad, DEG_W), jnp.float32),
            pltpu.SemaphoreType.DMA,
            pltpu.SemaphoreType.DMA,
        ])


def _agg_call(n, n_pad, e_pad, d):
    epw = e_pad // NW
    n_chunks = epw // ACHUNK
    offs, sizes = _row_split(n, NS)

    def body(z_hbm, src_hbm, dst_hbm, zeros_hbm, out,
             src_v0, src_v1, src_v2, src_v3, dst_v0, dst_v1, dst_v2, dst_v3,
             rows_v0, rows_v1, rows_v2, rows_v3, acc_sh,
             gsem0, gsem1, gsem2, gsem3):
        c = lax.axis_index("c")
        s = lax.axis_index("s")
        wid = s * NC + c

        for t in range(NS):
            if sizes[t] > 0:
                @pl.when((s == t) & (c == 0))
                def _():
                    pltpu.sync_copy(z_hbm.at[pl.ds(offs[t], sizes[t])],
                                    acc_sh.at[pl.ds(offs[t], sizes[t])])

                @pl.when((s == t) & (c != 0))
                def _():
                    pltpu.sync_copy(
                        zeros_hbm.at[pl.ds(offs[t], sizes[t])],
                        acc_sh.at[pl.ds(offs[t], sizes[t])])

        plsc.subcore_barrier()
        base = wid * epw

        srcs = [src_v0, src_v1, src_v2, src_v3]
        dsts = [dst_v0, dst_v1, dst_v2, dst_v3]
        rows = [rows_v0, rows_v1, rows_v2, rows_v3]
        gsems = [gsem0, gsem1, gsem2, gsem3]
        for k in range(DEPTH):
            pltpu.sync_copy(src_hbm.at[pl.ds(base + k * ACHUNK, ACHUNK)],
                            srcs[k])
            pltpu.sync_copy(dst_hbm.at[pl.ds(base + k * ACHUNK, ACHUNK)],
                            dsts[k])
            pltpu.async_copy(z_hbm.at[srcs[k]], rows[k], gsems[k])

        def step(j, carry):
            for k in range(DEPTH):
                i = DEPTH * j + k
                pltpu.make_async_copy(z_hbm.at[srcs[k]], rows[k],
                                      gsems[k]).wait()
                pltpu.sync_copy(rows[k], acc_sh.at[dsts[k]], add=True)

                @pl.when(i + DEPTH < n_chunks)
                def _():
                    off2 = base + (i + DEPTH) * ACHUNK
                    pltpu.sync_copy(src_hbm.at[pl.ds(off2, ACHUNK)], srcs[k])
                    pltpu.sync_copy(dst_hbm.at[pl.ds(off2, ACHUNK)], dsts[k])
                    pltpu.async_copy(z_hbm.at[srcs[k]], rows[k], gsems[k])
            return carry

        lax.fori_loop(0, n_chunks // DEPTH, step, 0)
        plsc.subcore_barrier()
        for t in range(NS):
            if sizes[t] > 0:
                @pl.when(s == t)
                def _():
                    pltpu.sync_copy(acc_sh.at[pl.ds(offs[t], sizes[t])],
                                    out.at[c, pl.ds(offs[t], sizes[t])])

    return pl.kernel(
        body,
        out_type=jax.ShapeDtypeStruct((NC, n, d), jnp.float32),
        mesh=_mesh(),
        compiler_params=_SC_PARAMS,
        scratch_types=(
            [pltpu.VMEM((ACHUNK,), jnp.int32)] * (2 * DEPTH)
            + [pltpu.VMEM((ACHUNK, d), jnp.float32)] * DEPTH
            + [pltpu.VMEM_SHARED((n_pad, d), jnp.float32)]
            + [pltpu.SemaphoreType.DMA] * DEPTH
        ))


def _lin1_call(n, d_in, d_h, rows):
    def body(parts_ref, x_ref, w_ref, z_ref, dis_ref):
        deg = jnp.sum(parts_ref[0] + parts_ref[1], axis=1, keepdims=True) + 1.0
        dis = lax.rsqrt(deg)
        z_ref[...] = dis * jnp.dot(x_ref[...], w_ref[...],
                                   preferred_element_type=jnp.float32)
        dis_ref[...] = dis

    return pl.pallas_call(
        body,
        grid=(n // rows,),
        in_specs=[
            pl.BlockSpec((NC, rows, DEG_W), lambda i: (0, i, 0)),
            pl.BlockSpec((rows, d_in), lambda i: (i, 0)),
            pl.BlockSpec((d_in, d_h), lambda i: (0, 0)),
        ],
        out_specs=[
            pl.BlockSpec((rows, d_h), lambda i: (i, 0)),
            pl.BlockSpec((rows, 1), lambda i: (i, 0)),
        ],
        out_shape=[
            jax.ShapeDtypeStruct((n, d_h), jnp.float32),
            jax.ShapeDtypeStruct((n, 1), jnp.float32),
        ])


def _mid_call(n, d_h, d2, rows):
    def body(acc_ref, dis_ref, b_ref, w_ref, z2_ref):
        dis = dis_ref[...]
        h = jnp.maximum(dis * (acc_ref[0] + acc_ref[1]) + b_ref[...], 0.0)
        z2_ref[...] = dis * jnp.dot(h, w_ref[...],
                                    preferred_element_type=jnp.float32)

    return pl.pallas_call(
        body,
        grid=(n // rows,),
        in_specs=[
            pl.BlockSpec((NC, rows, d_h), lambda i: (0, i, 0)),
            pl.BlockSpec((rows, 1), lambda i: (i, 0)),
            pl.BlockSpec((1, d_h), lambda i: (0, 0)),
            pl.BlockSpec((d_h, d2), lambda i: (0, 0)),
        ],
        out_specs=pl.BlockSpec((rows, d2), lambda i: (i, 0)),
        out_shape=jax.ShapeDtypeStruct((n, d2), jnp.float32))


def _final_call(n, d2, d_out, rows):
    def body(acc_ref, dis_ref, b_ref, w_ref, bl_ref, out_ref):
        dis = dis_ref[...]
        h = jnp.maximum(dis * (acc_ref[0] + acc_ref[1]) + b_ref[...], 0.0)
        out_ref[...] = jnp.dot(h, w_ref[...],
                               preferred_element_type=jnp.float32) + bl_ref[...]

    return pl.pallas_call(
        body,
        grid=(n // rows,),
        in_specs=[
            pl.BlockSpec((NC, rows, d2), lambda i: (0, i, 0)),
            pl.BlockSpec((rows, 1), lambda i: (i, 0)),
            pl.BlockSpec((1, d2), lambda i: (0, 0)),
            pl.BlockSpec((d2, d_out), lambda i: (0, 0)),
            pl.BlockSpec((1, d_out), lambda i: (0, 0)),
        ],
        out_specs=pl.BlockSpec((rows, d_out), lambda i: (i, 0)),
        out_shape=jax.ShapeDtypeStruct((n, d_out), jnp.float32))


def kernel(x, edge_index, W1, b1, W2, b2, Wl, bl):
    n, d_in = x.shape
    d_h = W1.shape[1]
    d2 = W2.shape[1]
    d_out = Wl.shape[1]
    e = edge_index.shape[1]
    rows = 1000

    group = max(DEPTH * ACHUNK, 2 * CHUNK)
    epw = -(-e // (NW * group)) * group
    e_pad = epw * NW
    pad = e_pad - e
    ei = edge_index.astype(jnp.int32)
    src_p = jnp.concatenate([ei[0], jnp.zeros((pad,), jnp.int32)])
    dst_p = jnp.concatenate([ei[1], jnp.full((pad,), n, jnp.int32)])
    n_pad = (n // 256 + 1) * 256

    d2p = d_h
    W2p = jnp.pad(W2, ((0, 0), (0, d2p - d2)))
    b2p = jnp.pad(b2, (0, d2p - d2)).reshape(1, d2p)
    Wlp = jnp.pad(Wl, ((0, d2p - d2), (0, 0)))

    zeros_h1 = jnp.zeros((n_pad, d_h), jnp.float32)
    zeros_deg = jnp.zeros((n_pad, DEG_W), jnp.float32)
    ones_small = jnp.full((CHUNK, DEG_W), 1.0 / DEG_W, jnp.float32)

    deg_parts = _deg_call(n, n_pad, e_pad)(dst_p, zeros_deg, ones_small)
    z1, dis = _lin1_call(n, d_in, d_h, rows)(deg_parts, x, W1)
    agg = _agg_call(n, n_pad, e_pad, d_h)
    acc1 = agg(z1, src_p, dst_p, zeros_h1)
    z2 = _mid_call(n, d_h, d2p, rows)(acc1, dis, b1.reshape(1, d_h), W2p)
    acc2 = agg(z2, src_p, dst_p, zeros_h1)
    out = _final_call(n, d2p, d_out, rows)(
        acc2, dis, b2p, Wlp, bl.reshape(1, d_out))
    return out


# speedup vs baseline: 1.9439x; 1.9439x over previous
import jax
import jax.numpy as jnp
from jax import lax
from jax.experimental import pallas as pl
from jax.experimental.pallas import tpu as pltpu
from jax.experimental.pallas import tpu_sc as plsc

NC = 2
NS = 16
NW = NC * NS
CHUNK = 128
DEPTH = 4
ACHUNK = 128
DEG_W = 16


_SC_PARAMS = pltpu.CompilerParams(use_tc_tiling_on_sc=False)


def _mesh():
    return plsc.VectorSubcoreMesh(
        core_axis_name="c", subcore_axis_name="s",
        num_cores=NC, num_subcores=NS)


def _row_split(n, n_tiles):
    base = (-(-n // n_tiles) + 15) // 16 * 16
    offs, sizes = [], []
    for t in range(n_tiles):
        o = min(t * base, n)
        sz = min(base, n - o)
        offs.append(o)
        sizes.append(sz)
    return offs, sizes


def _deg_call(n, n_pad, e_pad):
    epw = e_pad // NW
    n_chunks = epw // CHUNK
    zrpt = n_pad // NS
    offs, sizes = _row_split(n, NS)

    def body(dst_hbm, zeros_hbm, ones_hbm, out,
             dst_v0, dst_v1, ones_v, deg_sh, isem0, isem1):
        c = lax.axis_index("c")
        s = lax.axis_index("s")
        wid = s * NC + c
        pltpu.sync_copy(
            zeros_hbm.at[pl.ds(s * zrpt, zrpt)],
            deg_sh.at[pl.ds(s * zrpt, zrpt)])
        pltpu.sync_copy(ones_hbm, ones_v)
        plsc.subcore_barrier()
        base = wid * epw

        pltpu.sync_copy(dst_hbm.at[pl.ds(base, CHUNK)], dst_v0)
        pltpu.async_copy(dst_hbm.at[pl.ds(base + CHUNK, CHUNK)], dst_v1, isem1)

        def half(i, dst_c, dst_n, isem_c, isem_n):
            @pl.when(i + 1 < n_chunks)
            def _():
                pltpu.make_async_copy(
                    dst_hbm.at[pl.ds(0, CHUNK)], dst_n, isem_n).wait()

            @pl.when(i + 2 < n_chunks)
            def _():
                pltpu.async_copy(
                    dst_hbm.at[pl.ds(base + (i + 2) * CHUNK, CHUNK)],
                    dst_c, isem_c)

            pltpu.sync_copy(ones_v, deg_sh.at[dst_c], add=True)

        def step(j, carry):
            half(2 * j, dst_v0, dst_v1, isem0, isem1)
            half(2 * j + 1, dst_v1, dst_v0, isem1, isem0)
            return carry

        lax.fori_loop(0, n_chunks // 2, step, 0)
        plsc.subcore_barrier()
        for t in range(NS):
            if sizes[t] > 0:
                @pl.when(s == t)
                def _():
                    pltpu.sync_copy(
                        deg_sh.at[pl.ds(offs[t], sizes[t])],
                        out.at[c, pl.ds(offs[t], sizes[t])])

    return pl.kernel(
        body,
        out_type=jax.ShapeDtypeStruct((NC, n, DEG_W), jnp.float32),
        mesh=_mesh(),
        compiler_params=_SC_PARAMS,
        scratch_types=[
            pltpu.VMEM((CHUNK,), jnp.int32),
            pltpu.VMEM((CHUNK,), jnp.int32),
            pltpu.VMEM((CHUNK, DEG_W), jnp.float32),
            pltpu.VMEM_SHARED((n_pad, DEG_W), jnp.float32),
            pltpu.SemaphoreType.DMA,
            pltpu.SemaphoreType.DMA,
        ])


def _agg_call(n, n_pad, e_pad):
    epw = e_pad // NW
    n_chunks = epw // ACHUNK
    offs, sizes = _row_split(n, NS)
    D = 64

    def body(z_hbm, src_hbm, dst_hbm, zeros_hbm, out,
             src_v0, src_v1, dst_v0, dst_v1, rows_v0, rows_v1,
             z_sh, acc_sh, gsem0, gsem1):
        c = lax.axis_index("c")
        s = lax.axis_index("s")
        wid = s * NC + c

        for t in range(NS):
            if sizes[t] > 0:
                sl = pl.ds(offs[t], sizes[t])

                @pl.when(s == t)
                def _():
                    pltpu.sync_copy(z_hbm.at[sl], z_sh.at[sl])

                @pl.when((s == t) & (c == 0))
                def _():
                    pltpu.sync_copy(z_hbm.at[sl], acc_sh.at[sl])

                @pl.when((s == t) & (c != 0))
                def _():
                    pltpu.sync_copy(zeros_hbm.at[sl], acc_sh.at[sl])

        plsc.subcore_barrier()
        base = wid * epw

        srcs = [src_v0, src_v1]
        dsts = [dst_v0, dst_v1]
        rows = [rows_v0, rows_v1]
        gsems = [gsem0, gsem1]
        for k in range(2):
            pltpu.sync_copy(src_hbm.at[pl.ds(base + k * ACHUNK, ACHUNK)],
                            srcs[k])
            pltpu.sync_copy(dst_hbm.at[pl.ds(base + k * ACHUNK, ACHUNK)],
                            dsts[k])
            pltpu.async_copy(z_sh.at[srcs[k]], rows[k], gsems[k])

        def step(j, carry):
            for k in range(2):
                i = 2 * j + k
                pltpu.make_async_copy(z_sh.at[srcs[k]], rows[k],
                                      gsems[k]).wait()
                pltpu.sync_copy(rows[k], acc_sh.at[dsts[k]], add=True)

                @pl.when(i + 2 < n_chunks)
                def _():
                    off2 = base + (i + 2) * ACHUNK
                    pltpu.sync_copy(src_hbm.at[pl.ds(off2, ACHUNK)], srcs[k])
                    pltpu.sync_copy(dst_hbm.at[pl.ds(off2, ACHUNK)], dsts[k])
                    pltpu.async_copy(z_sh.at[srcs[k]], rows[k], gsems[k])
            return carry

        lax.fori_loop(0, n_chunks // 2, step, 0)
        plsc.subcore_barrier()
        for t in range(NS):
            if sizes[t] > 0:
                @pl.when(s == t)
                def _():
                    pltpu.sync_copy(acc_sh.at[pl.ds(offs[t], sizes[t])],
                                    out.at[c, pl.ds(offs[t], sizes[t])])

    return pl.kernel(
        body,
        out_type=jax.ShapeDtypeStruct((NC, n, D), jnp.float32),
        mesh=_mesh(),
        compiler_params=_SC_PARAMS,
        scratch_types=(
            [pltpu.VMEM((ACHUNK,), jnp.int32)] * 4
            + [pltpu.VMEM((ACHUNK, D), jnp.float32)] * 2
            + [pltpu.VMEM_SHARED((n_pad, D), jnp.float32)] * 2
            + [pltpu.SemaphoreType.DMA] * 2
        ))


def _lin1_call(n, d_in, d_h, rows):
    def body(parts_ref, x_ref, w_ref, za_ref, zb_ref, dis_ref):
        deg = jnp.sum(parts_ref[0] + parts_ref[1], axis=1, keepdims=True) + 1.0
        dis = lax.rsqrt(deg)
        z = dis * jnp.dot(x_ref[...], w_ref[...],
                          preferred_element_type=jnp.float32)
        za_ref[...] = z[:, :d_h // 2]
        zb_ref[...] = z[:, d_h // 2:]
        dis_ref[...] = dis

    return pl.pallas_call(
        body,
        grid=(n // rows,),
        in_specs=[
            pl.BlockSpec((NC, rows, DEG_W), lambda i: (0, i, 0)),
            pl.BlockSpec((rows, d_in), lambda i: (i, 0)),
            pl.BlockSpec((d_in, d_h), lambda i: (0, 0)),
        ],
        out_specs=[
            pl.BlockSpec((rows, d_h // 2), lambda i: (i, 0)),
            pl.BlockSpec((rows, d_h // 2), lambda i: (i, 0)),
            pl.BlockSpec((rows, 1), lambda i: (i, 0)),
        ],
        out_shape=[
            jax.ShapeDtypeStruct((n, d_h // 2), jnp.float32),
            jax.ShapeDtypeStruct((n, d_h // 2), jnp.float32),
            jax.ShapeDtypeStruct((n, 1), jnp.float32),
        ])


def _mid_call(n, d_h, d2, rows):
    def body(acca_ref, accb_ref, dis_ref, b_ref, w_ref, z2_ref):
        dis = dis_ref[...]
        agg = jnp.concatenate(
            [acca_ref[0] + acca_ref[1], accb_ref[0] + accb_ref[1]], axis=1)
        h = jnp.maximum(dis * agg + b_ref[...], 0.0)
        z2_ref[...] = dis * jnp.dot(h, w_ref[...],
                                    preferred_element_type=jnp.float32)

    return pl.pallas_call(
        body,
        grid=(n // rows,),
        in_specs=[
            pl.BlockSpec((NC, rows, d_h // 2), lambda i: (0, i, 0)),
            pl.BlockSpec((NC, rows, d_h // 2), lambda i: (0, i, 0)),
            pl.BlockSpec((rows, 1), lambda i: (i, 0)),
            pl.BlockSpec((1, d_h), lambda i: (0, 0)),
            pl.BlockSpec((d_h, d2), lambda i: (0, 0)),
        ],
        out_specs=pl.BlockSpec((rows, d2), lambda i: (i, 0)),
        out_shape=jax.ShapeDtypeStruct((n, d2), jnp.float32))


def _final_call(n, d2, d_out, rows):
    def body(acc_ref, dis_ref, b_ref, w_ref, bl_ref, out_ref):
        dis = dis_ref[...]
        h = jnp.maximum(dis * (acc_ref[0] + acc_ref[1]) + b_ref[...], 0.0)
        out_ref[...] = jnp.dot(h, w_ref[...],
                               preferred_element_type=jnp.float32) + bl_ref[...]

    return pl.pallas_call(
        body,
        grid=(n // rows,),
        in_specs=[
            pl.BlockSpec((NC, rows, d2), lambda i: (0, i, 0)),
            pl.BlockSpec((rows, 1), lambda i: (i, 0)),
            pl.BlockSpec((1, d2), lambda i: (0, 0)),
            pl.BlockSpec((d2, d_out), lambda i: (0, 0)),
            pl.BlockSpec((1, d_out), lambda i: (0, 0)),
        ],
        out_specs=pl.BlockSpec((rows, d_out), lambda i: (i, 0)),
        out_shape=jax.ShapeDtypeStruct((n, d_out), jnp.float32))


def kernel(x, edge_index, W1, b1, W2, b2, Wl, bl):
    n, d_in = x.shape
    d_h = W1.shape[1]
    d2 = W2.shape[1]
    d_out = Wl.shape[1]
    e = edge_index.shape[1]
    rows = 1000

    group = max(DEPTH * ACHUNK, 2 * CHUNK)
    epw = -(-e // (NW * group)) * group
    e_pad = epw * NW
    pad = e_pad - e
    ei = edge_index.astype(jnp.int32)
    src_p = jnp.concatenate([ei[0], jnp.zeros((pad,), jnp.int32)])
    dst_p = jnp.concatenate([ei[1], jnp.full((pad,), n, jnp.int32)])
    n_pad = (n // 256 + 1) * 256

    zeros_64 = jnp.zeros((n_pad, 64), jnp.float32)
    zeros_deg = jnp.zeros((n_pad, DEG_W), jnp.float32)
    ones_small = jnp.full((CHUNK, DEG_W), 1.0 / DEG_W, jnp.float32)

    deg_parts = _deg_call(n, n_pad, e_pad)(dst_p, zeros_deg, ones_small)
    z1a, z1b, dis = _lin1_call(n, d_in, d_h, rows)(deg_parts, x, W1)
    agg = _agg_call(n, n_pad, e_pad)
    acc1a = agg(z1a, src_p, dst_p, zeros_64)
    acc1b = agg(z1b, src_p, dst_p, zeros_64)
    z2 = _mid_call(n, d_h, d2, rows)(
        acc1a, acc1b, dis, b1.reshape(1, d_h), W2)
    acc2 = agg(z2, src_p, dst_p, zeros_64)
    out = _final_call(n, d2, d_out, rows)(
        acc2, dis, b2.reshape(1, d2), Wl, bl.reshape(1, d_out))
    return out


# speedup vs baseline: 2.3339x; 1.2006x over previous
import jax
import jax.numpy as jnp
from jax import lax
from jax.experimental import pallas as pl
from jax.experimental.pallas import tpu as pltpu
from jax.experimental.pallas import tpu_sc as plsc

NC = 2
NS = 16
NW = NC * NS
CHUNK = 128
DEPTH = 4
ACHUNK = 128
DEG_W = 16


_SC_PARAMS = pltpu.CompilerParams(use_tc_tiling_on_sc=False)


def _mesh():
    return plsc.VectorSubcoreMesh(
        core_axis_name="c", subcore_axis_name="s",
        num_cores=NC, num_subcores=NS)


def _row_split(n, n_tiles):
    base = (-(-n // n_tiles) + 15) // 16 * 16
    offs, sizes = [], []
    for t in range(n_tiles):
        o = min(t * base, n)
        sz = min(base, n - o)
        offs.append(o)
        sizes.append(sz)
    return offs, sizes


def _deg_call(n, n_pad, e_pad):
    epw = e_pad // NW
    n_chunks = epw // CHUNK
    zrpt = n_pad // NS
    offs, sizes = _row_split(n, NS)

    def body(dst_hbm, zeros_hbm, ones_hbm, out,
             dst_v0, dst_v1, ones_v, deg_sh, isem0, isem1):
        c = lax.axis_index("c")
        s = lax.axis_index("s")
        wid = s * NC + c
        pltpu.sync_copy(
            zeros_hbm.at[pl.ds(s * zrpt, zrpt)],
            deg_sh.at[pl.ds(s * zrpt, zrpt)])
        pltpu.sync_copy(ones_hbm, ones_v)
        plsc.subcore_barrier()
        base = wid * epw

        pltpu.sync_copy(dst_hbm.at[pl.ds(base, CHUNK)], dst_v0)
        pltpu.async_copy(dst_hbm.at[pl.ds(base + CHUNK, CHUNK)], dst_v1, isem1)

        def half(i, dst_c, dst_n, isem_c, isem_n):
            @pl.when(i + 1 < n_chunks)
            def _():
                pltpu.make_async_copy(
                    dst_hbm.at[pl.ds(0, CHUNK)], dst_n, isem_n).wait()

            @pl.when(i + 2 < n_chunks)
            def _():
                pltpu.async_copy(
                    dst_hbm.at[pl.ds(base + (i + 2) * CHUNK, CHUNK)],
                    dst_c, isem_c)

            pltpu.sync_copy(ones_v, deg_sh.at[dst_c], add=True)

        def step(j, carry):
            half(2 * j, dst_v0, dst_v1, isem0, isem1)
            half(2 * j + 1, dst_v1, dst_v0, isem1, isem0)
            return carry

        lax.fori_loop(0, n_chunks // 2, step, 0)
        plsc.subcore_barrier()
        for t in range(NS):
            if sizes[t] > 0:
                @pl.when(s == t)
                def _():
                    pltpu.sync_copy(
                        deg_sh.at[pl.ds(offs[t], sizes[t])],
                        out.at[c, pl.ds(offs[t], sizes[t])])

    return pl.kernel(
        body,
        out_type=jax.ShapeDtypeStruct((NC, n, DEG_W), jnp.float32),
        mesh=_mesh(),
        compiler_params=_SC_PARAMS,
        scratch_types=[
            pltpu.VMEM((CHUNK,), jnp.int32),
            pltpu.VMEM((CHUNK,), jnp.int32),
            pltpu.VMEM((CHUNK, DEG_W), jnp.float32),
            pltpu.VMEM_SHARED((n_pad, DEG_W), jnp.float32),
            pltpu.SemaphoreType.DMA,
            pltpu.SemaphoreType.DMA,
        ])


def _agg_call(n, n_pad, e_pad):
    epw = e_pad // NW
    n_chunks = epw // ACHUNK
    offs, sizes = _row_split(n, NS)
    D = 64

    def body(z_hbm, src_hbm, dst_hbm, zeros_hbm, out,
             src_v0, src_v1, dst_v0, dst_v1, rows_v0, rows_v1,
             z_sh, acc_sh, gsem0, gsem1, isem0, isem1):
        c = lax.axis_index("c")
        s = lax.axis_index("s")
        wid = s * NC + c

        for t in range(NS):
            if sizes[t] > 0:
                sl = pl.ds(offs[t], sizes[t])

                @pl.when(s == t)
                def _():
                    pltpu.sync_copy(z_hbm.at[sl], z_sh.at[sl])

                @pl.when((s == t) & (c == 0))
                def _():
                    pltpu.sync_copy(z_hbm.at[sl], acc_sh.at[sl])

                @pl.when((s == t) & (c != 0))
                def _():
                    pltpu.sync_copy(zeros_hbm.at[sl], acc_sh.at[sl])

        plsc.subcore_barrier()
        base = wid * epw

        pltpu.sync_copy(src_hbm.at[pl.ds(base, ACHUNK)], src_v0)
        pltpu.sync_copy(dst_hbm.at[pl.ds(base, ACHUNK)], dst_v0)
        pltpu.async_copy(z_sh.at[src_v0], rows_v0, gsem0)
        pltpu.async_copy(src_hbm.at[pl.ds(base + ACHUNK, ACHUNK)],
                         src_v1, isem1)
        pltpu.async_copy(dst_hbm.at[pl.ds(base + ACHUNK, ACHUNK)],
                         dst_v1, isem1)

        def half(i, src_c, dst_c, rows_c, gsem_c, isem_c,
                 src_n, dst_n, rows_n, gsem_n, isem_n):
            @pl.when(i + 1 < n_chunks)
            def _():
                pltpu.make_async_copy(
                    src_hbm.at[pl.ds(0, ACHUNK)], src_n, isem_n).wait()
                pltpu.make_async_copy(
                    dst_hbm.at[pl.ds(0, ACHUNK)], dst_n, isem_n).wait()
                pltpu.async_copy(z_sh.at[src_n], rows_n, gsem_n)

            pltpu.make_async_copy(z_sh.at[src_c], rows_c, gsem_c).wait()
            pltpu.sync_copy(rows_c, acc_sh.at[dst_c], add=True)

            @pl.when(i + 2 < n_chunks)
            def _():
                off2 = base + (i + 2) * ACHUNK
                pltpu.async_copy(src_hbm.at[pl.ds(off2, ACHUNK)],
                                 src_c, isem_c)
                pltpu.async_copy(dst_hbm.at[pl.ds(off2, ACHUNK)],
                                 dst_c, isem_c)

        def step(j, carry):
            half(2 * j, src_v0, dst_v0, rows_v0, gsem0, isem0,
                 src_v1, dst_v1, rows_v1, gsem1, isem1)
            half(2 * j + 1, src_v1, dst_v1, rows_v1, gsem1, isem1,
                 src_v0, dst_v0, rows_v0, gsem0, isem0)
            return carry

        lax.fori_loop(0, n_chunks // 2, step, 0)
        plsc.subcore_barrier()
        for t in range(NS):
            if sizes[t] > 0:
                @pl.when(s == t)
                def _():
                    pltpu.sync_copy(acc_sh.at[pl.ds(offs[t], sizes[t])],
                                    out.at[c, pl.ds(offs[t], sizes[t])])

    return pl.kernel(
        body,
        out_type=jax.ShapeDtypeStruct((NC, n, D), jnp.float32),
        mesh=_mesh(),
        compiler_params=_SC_PARAMS,
        scratch_types=(
            [pltpu.VMEM((ACHUNK,), jnp.int32)] * 4
            + [pltpu.VMEM((ACHUNK, D), jnp.float32)] * 2
            + [pltpu.VMEM_SHARED((n_pad, D), jnp.float32)] * 2
            + [pltpu.SemaphoreType.DMA] * 4
        ))


def _lin1_call(n, d_in, d_h, rows):
    def body(parts_ref, x_ref, w_ref, za_ref, zb_ref, dis_ref):
        deg = jnp.sum(parts_ref[0] + parts_ref[1], axis=1, keepdims=True) + 1.0
        dis = lax.rsqrt(deg)
        z = dis * jnp.dot(x_ref[...], w_ref[...],
                          preferred_element_type=jnp.float32)
        za_ref[...] = z[:, :d_h // 2]
        zb_ref[...] = z[:, d_h // 2:]
        dis_ref[...] = dis

    return pl.pallas_call(
        body,
        grid=(n // rows,),
        in_specs=[
            pl.BlockSpec((NC, rows, DEG_W), lambda i: (0, i, 0)),
            pl.BlockSpec((rows, d_in), lambda i: (i, 0)),
            pl.BlockSpec((d_in, d_h), lambda i: (0, 0)),
        ],
        out_specs=[
            pl.BlockSpec((rows, d_h // 2), lambda i: (i, 0)),
            pl.BlockSpec((rows, d_h // 2), lambda i: (i, 0)),
            pl.BlockSpec((rows, 1), lambda i: (i, 0)),
        ],
        out_shape=[
            jax.ShapeDtypeStruct((n, d_h // 2), jnp.float32),
            jax.ShapeDtypeStruct((n, d_h // 2), jnp.float32),
            jax.ShapeDtypeStruct((n, 1), jnp.float32),
        ])


def _mid_call(n, d_h, d2, rows):
    def body(acca_ref, accb_ref, dis_ref, b_ref, w_ref, z2_ref):
        dis = dis_ref[...]
        agg = jnp.concatenate(
            [acca_ref[0] + acca_ref[1], accb_ref[0] + accb_ref[1]], axis=1)
        h = jnp.maximum(dis * agg + b_ref[...], 0.0)
        z2_ref[...] = dis * jnp.dot(h, w_ref[...],
                                    preferred_element_type=jnp.float32)

    return pl.pallas_call(
        body,
        grid=(n // rows,),
        in_specs=[
            pl.BlockSpec((NC, rows, d_h // 2), lambda i: (0, i, 0)),
            pl.BlockSpec((NC, rows, d_h // 2), lambda i: (0, i, 0)),
            pl.BlockSpec((rows, 1), lambda i: (i, 0)),
            pl.BlockSpec((1, d_h), lambda i: (0, 0)),
            pl.BlockSpec((d_h, d2), lambda i: (0, 0)),
        ],
        out_specs=pl.BlockSpec((rows, d2), lambda i: (i, 0)),
        out_shape=jax.ShapeDtypeStruct((n, d2), jnp.float32))


def _final_call(n, d2, d_out, rows):
    def body(acc_ref, dis_ref, b_ref, w_ref, bl_ref, out_ref):
        dis = dis_ref[...]
        h = jnp.maximum(dis * (acc_ref[0] + acc_ref[1]) + b_ref[...], 0.0)
        out_ref[...] = jnp.dot(h, w_ref[...],
                               preferred_element_type=jnp.float32) + bl_ref[...]

    return pl.pallas_call(
        body,
        grid=(n // rows,),
        in_specs=[
            pl.BlockSpec((NC, rows, d2), lambda i: (0, i, 0)),
            pl.BlockSpec((rows, 1), lambda i: (i, 0)),
            pl.BlockSpec((1, d2), lambda i: (0, 0)),
            pl.BlockSpec((d2, d_out), lambda i: (0, 0)),
            pl.BlockSpec((1, d_out), lambda i: (0, 0)),
        ],
        out_specs=pl.BlockSpec((rows, d_out), lambda i: (i, 0)),
        out_shape=jax.ShapeDtypeStruct((n, d_out), jnp.float32))


def kernel(x, edge_index, W1, b1, W2, b2, Wl, bl):
    n, d_in = x.shape
    d_h = W1.shape[1]
    d2 = W2.shape[1]
    d_out = Wl.shape[1]
    e = edge_index.shape[1]
    rows = 1000

    group = max(DEPTH * ACHUNK, 2 * CHUNK)
    epw = -(-e // (NW * group)) * group
    e_pad = epw * NW
    pad = e_pad - e
    ei = edge_index.astype(jnp.int32)
    src_p = jnp.concatenate([ei[0], jnp.zeros((pad,), jnp.int32)])
    dst_p = jnp.concatenate([ei[1], jnp.full((pad,), n, jnp.int32)])
    n_pad = (n // 256 + 1) * 256

    zeros_64 = jnp.zeros((n_pad, 64), jnp.float32)
    zeros_deg = jnp.zeros((n_pad, DEG_W), jnp.float32)
    ones_small = jnp.full((CHUNK, DEG_W), 1.0 / DEG_W, jnp.float32)

    deg_parts = _deg_call(n, n_pad, e_pad)(dst_p, zeros_deg, ones_small)
    z1a, z1b, dis = _lin1_call(n, d_in, d_h, rows)(deg_parts, x, W1)
    agg = _agg_call(n, n_pad, e_pad)
    acc1a = agg(z1a, src_p, dst_p, zeros_64)
    acc1b = agg(z1b, src_p, dst_p, zeros_64)
    z2 = _mid_call(n, d_h, d2, rows)(
        acc1a, acc1b, dis, b1.reshape(1, d_h), W2)
    acc2 = agg(z2, src_p, dst_p, zeros_64)
    out = _final_call(n, d2, d_out, rows)(
        acc2, dis, b2.reshape(1, d2), Wl, bl.reshape(1, d_out))
    return out
